# speedup vs baseline: 2.8666x; 2.8666x over previous
import jax
import jax.numpy as jnp
from jax import lax
from jax.experimental import pallas as pl
from jax.experimental.pallas import tpu as pltpu
from jax.experimental.pallas import tpu_sc as plsc

N = 10000
E = 320000
D = 128
B = 16

NC = 2
NS = 16
NW = NC * NS

N_PAD = 10240
ROWS_G = N_PAD // NW
GCH = 80
E_TILE = 10240
E_PAD = E_TILE * NW
ECH = 128
N_ECH = E_TILE // ECH
ROWS_SC = N_PAD // NS
ZR = 128

TBLK = 2048
NB = N_PAD // TBLK


def _sc_gather_body(ids_hbm, table_hbm, out_hbm, idx_v, rows_v, sem):
    c = lax.axis_index("c")
    s = lax.axis_index("s")
    wid = s * NC + c

    def chunk(i, carry):
        base = wid * ROWS_G + i * GCH
        pltpu.sync_copy(ids_hbm.at[pl.ds(base, GCH)], idx_v)
        pltpu.async_copy(table_hbm.at[idx_v], rows_v, sem).wait()
        pltpu.sync_copy(rows_v, out_hbm.at[pl.ds(base, GCH)])
        return carry

    lax.fori_loop(0, ROWS_G // GCH, chunk, 0)


def _sc_scatter_body(src_hbm, dst_hbm, x_hbm, zeros_hbm, out_hbm,
                     idx_s, idx_d, rows_v, zbuf, shared, sem):
    c = lax.axis_index("c")
    s = lax.axis_index("s")
    wid = s * NC + c

    pltpu.sync_copy(zeros_hbm, zbuf)
    row0 = s * ROWS_SC

    def zloop(k, carry):
        pltpu.sync_copy(zbuf, shared.at[pl.ds(row0 + k * ZR, ZR)])
        return carry

    lax.fori_loop(0, ROWS_SC // ZR, zloop, 0)
    plsc.subcore_barrier()

    def echunk(i, carry):
        base = wid * E_TILE + i * ECH
        pltpu.sync_copy(src_hbm.at[pl.ds(base, ECH)], idx_s)
        pltpu.sync_copy(dst_hbm.at[pl.ds(base, ECH)], idx_d)
        pltpu.async_copy(x_hbm.at[idx_s], rows_v, sem).wait()
        pltpu.sync_copy(rows_v, shared.at[idx_d], add=True)
        return carry

    lax.fori_loop(0, N_ECH, echunk, 0)
    plsc.subcore_barrier()
    pltpu.sync_copy(shared.at[pl.ds(row0, ROWS_SC)],
                    out_hbm.at[c, pl.ds(row0, ROWS_SC)])


def _tc_layer_body(x_ref, agg_ref, w_ref, b_ref, out_ref):
    h = x_ref[...] + agg_ref[0] + agg_ref[1]
    y = jnp.dot(h, w_ref[...], preferred_element_type=jnp.float32) + b_ref[...]
    out_ref[...] = jnp.maximum(y, 0.0)


def _tc_pool_body(x_ref, seg_ref, wf_ref, bf_ref, out_ref, sums, counts):
    i = pl.program_id(0)

    @pl.when(i == 0)
    def _():
        sums[...] = jnp.zeros_like(sums)
        counts[...] = jnp.zeros_like(counts)

    seg = seg_ref[0, 0, :]
    mask = (seg[:, None] == lax.broadcasted_iota(jnp.int32, (1, B), 1)
            ).astype(jnp.float32)
    dn = (((0,), (0,)), ((), ()))
    sums[...] += lax.dot_general(mask, x_ref[...], dn,
                                 preferred_element_type=jnp.float32)
    counts[...] += lax.dot_general(mask, jnp.ones_like(x_ref[...]), dn,
                                   preferred_element_type=jnp.float32)

    @pl.when(i == pl.num_programs(0) - 1)
    def _():
        pooled = sums[...] / jnp.maximum(counts[...], 1.0)
        out_ref[...] = jnp.dot(pooled, wf_ref[...],
                               preferred_element_type=jnp.float32) + bf_ref[...]


def _make_sc_gather():
    mesh = plsc.VectorSubcoreMesh(core_axis_name="c", subcore_axis_name="s")
    return pl.kernel(
        _sc_gather_body,
        out_type=jax.ShapeDtypeStruct((N_PAD, D), jnp.float32),
        mesh=mesh,
        scratch_types=[
            pltpu.VMEM((GCH,), jnp.int32),
            pltpu.VMEM((GCH, D), jnp.float32),
            pltpu.SemaphoreType.DMA,
        ],
    )


def _make_sc_scatter():
    mesh = plsc.VectorSubcoreMesh(core_axis_name="c", subcore_axis_name="s")
    return pl.kernel(
        _sc_scatter_body,
        out_type=jax.ShapeDtypeStruct((NC, N_PAD, D), jnp.float32),
        mesh=mesh,
        scratch_types=[
            pltpu.VMEM((ECH,), jnp.int32),
            pltpu.VMEM((ECH,), jnp.int32),
            pltpu.VMEM((ECH, D), jnp.float32),
            pltpu.VMEM((ZR, D), jnp.float32),
            pltpu.VMEM_SHARED((N_PAD, D), jnp.float32),
            pltpu.SemaphoreType.DMA,
        ],
    )


def _tc_layer(x, agg, w, b):
    return pl.pallas_call(
        _tc_layer_body,
        grid=(NB,),
        in_specs=[
            pl.BlockSpec((TBLK, D), lambda i: (i, 0)),
            pl.BlockSpec((NC, TBLK, D), lambda i: (0, i, 0)),
            pl.BlockSpec((D, D), lambda i: (0, 0)),
            pl.BlockSpec((1, D), lambda i: (0, 0)),
        ],
        out_specs=pl.BlockSpec((TBLK, D), lambda i: (i, 0)),
        out_shape=jax.ShapeDtypeStruct((N_PAD, D), jnp.float32),
    )(x, agg, w, b.reshape(1, D))


def _tc_pool(x, seg3, wf, bf):
    return pl.pallas_call(
        _tc_pool_body,
        grid=(NB,),
        in_specs=[
            pl.BlockSpec((TBLK, D), lambda i: (i, 0)),
            pl.BlockSpec((1, 1, TBLK), lambda i: (i, 0, 0)),
            pl.BlockSpec((D, D), lambda i: (0, 0)),
            pl.BlockSpec((1, D), lambda i: (0, 0)),
        ],
        out_specs=pl.BlockSpec((B, D), lambda i: (0, 0)),
        out_shape=jax.ShapeDtypeStruct((B, D), jnp.float32),
        scratch_shapes=[
            pltpu.VMEM((B, D), jnp.float32),
            pltpu.VMEM((B, D), jnp.float32),
        ],
    )(x, seg3, wf, bf.reshape(1, D))


def kernel(node_ids, edge_index, batch, emb, W1, b1, W2, b2, W3, b3, Wf, bf):
    node_ids_p = jnp.concatenate(
        [node_ids.astype(jnp.int32), jnp.zeros((N_PAD - N,), jnp.int32)])
    src_p = jnp.concatenate(
        [edge_index[0].astype(jnp.int32), jnp.zeros((E_PAD - E,), jnp.int32)])
    dst_p = jnp.concatenate(
        [edge_index[1].astype(jnp.int32), jnp.full((E_PAD - E,), N, jnp.int32)])
    zeros_blk = jnp.zeros((ZR, D), jnp.float32)
    seg3 = jnp.concatenate(
        [batch.astype(jnp.int32), jnp.full((N_PAD - N,), B, jnp.int32)]
    ).reshape(NB, 1, TBLK)

    sc_gather = _make_sc_gather()
    sc_scatter = _make_sc_scatter()

    x = sc_gather(node_ids_p, emb)
    for w, b in ((W1, b1), (W2, b2), (W3, b3)):
        agg = sc_scatter(src_p, dst_p, x, zeros_blk)
        x = _tc_layer(x, agg, w, b)
    return _tc_pool(x, seg3, Wf, bf)
